# Initial kernel scaffold; baseline (speedup 1.0000x reference)
import functools

import jax
import jax.numpy as jnp
from jax.experimental import pallas as pl
from jax.experimental.pallas import tpu as pltpu

EMB = 64
VW = 44
VWP = 48
M1, M2 = 11, 10
TOK_BLOCK = 512


def _gather_mean_body(idx_ref, wt_ref, table_ref, out_ref, acc_ref, *, M):
    w = pl.program_id(0)
    j = pl.program_id(1)

    @pl.when(j == 0)
    def _():
        acc_ref[...] = jnp.zeros_like(acc_ref)

    acc_ref[...] += wt_ref[w * M + j] * table_ref[...]

    @pl.when(j == M - 1)
    def _():
        out_ref[...] = acc_ref[...]


def _gather_mean(table, idx, mask, cnt, M):
    wt = (mask.astype(jnp.float32) / cnt.astype(jnp.float32)[:, None])
    wtp = jnp.zeros((VWP, M), jnp.float32).at[:VW].set(wt).reshape(-1)
    idxp = jnp.zeros((VWP, M), jnp.int32).at[:VW].set(idx).reshape(-1)

    grid_spec = pltpu.PrefetchScalarGridSpec(
        num_scalar_prefetch=2,
        grid=(VWP, M),
        in_specs=[pl.BlockSpec((1, EMB), lambda w, j, idx_s, wt_s: (idx_s[w * M + j], 0))],
        out_specs=pl.BlockSpec((1, EMB), lambda w, j, idx_s, wt_s: (w, 0)),
        scratch_shapes=[pltpu.VMEM((1, EMB), jnp.float32)],
    )
    return pl.pallas_call(
        functools.partial(_gather_mean_body, M=M),
        grid_spec=grid_spec,
        out_shape=jax.ShapeDtypeStruct((VWP, EMB), jnp.float32),
    )(idxp, wtp, table)


def _lookup_body(x_ref, e1_ref, e2_ref, t0_ref, out_ref):
    xb = x_ref[0, 0, :]
    oh = (xb[:, None] == jax.lax.broadcasted_iota(jnp.int32, (TOK_BLOCK, VWP), 1)
          ).astype(jnp.float32)
    w1 = jnp.dot(oh, e1_ref[...], preferred_element_type=jnp.float32)
    w2 = jnp.dot(oh, e2_ref[...], preferred_element_type=jnp.float32)
    word = jnp.concatenate([w1, w2], axis=1)
    xs = jnp.minimum(xb, 3)
    ohs = (xs[:, None] == jax.lax.broadcasted_iota(jnp.int32, (TOK_BLOCK, 8), 1)
           ).astype(jnp.float32)
    spec = jnp.dot(ohs, t0_ref[...], preferred_element_type=jnp.float32)
    out_ref[...] = jnp.where((xb < 4)[:, None], spec, word)


def kernel(x, table0, table1, table2, idx1, mask1, cnt1, idx2, mask2, cnt2):
    B, L = x.shape
    N = B * L
    nblk = N // TOK_BLOCK

    e1 = _gather_mean(table1, idx1, mask1, cnt1, M1)
    e2 = _gather_mean(table2, idx2, mask2, cnt2, M2)
    t0p = jnp.zeros((8, 2 * EMB), jnp.float32).at[:4].set(table0)
    x3 = x.reshape(nblk, 1, TOK_BLOCK)

    out = pl.pallas_call(
        _lookup_body,
        grid=(nblk,),
        in_specs=[
            pl.BlockSpec((1, 1, TOK_BLOCK), lambda i: (i, 0, 0)),
            pl.BlockSpec((VWP, EMB), lambda i: (0, 0)),
            pl.BlockSpec((VWP, EMB), lambda i: (0, 0)),
            pl.BlockSpec((8, 2 * EMB), lambda i: (0, 0)),
        ],
        out_specs=pl.BlockSpec((TOK_BLOCK, 2 * EMB), lambda i: (i, 0)),
        out_shape=jax.ShapeDtypeStruct((N, 2 * EMB), jnp.float32),
    )(x3, e1, e2, t0p)

    return out.reshape(B, L, 2 * EMB)


# speedup vs baseline: 10.6259x; 10.6259x over previous
import functools

import jax
import jax.numpy as jnp
from jax.experimental import pallas as pl
from jax.experimental.pallas import tpu as pltpu

EMB = 64
VW = 44
VWP = 48
M1, M2 = 11, 10
TOK_BLOCK = 512


def _gather_mean_body(idx_ref, wt_ref, table_ref, out_ref, acc_ref, *, M):
    w = pl.program_id(0)
    j = pl.program_id(1)

    @pl.when(j == 0)
    def _():
        acc_ref[...] = jnp.zeros_like(acc_ref)

    acc_ref[...] += wt_ref[w * M + j] * table_ref[...]

    @pl.when(j == M - 1)
    def _():
        out_ref[...] = acc_ref[...]


def _gather_mean(table, idx, mask, cnt, M):
    wt = (mask.astype(jnp.float32) / cnt.astype(jnp.float32)[:, None])
    wtp = jnp.zeros((VWP, M), jnp.float32).at[:VW].set(wt).reshape(-1)
    idxp = jnp.zeros((VWP, M), jnp.int32).at[:VW].set(idx).reshape(-1)

    grid_spec = pltpu.PrefetchScalarGridSpec(
        num_scalar_prefetch=2,
        grid=(VWP, M),
        in_specs=[pl.BlockSpec((1, 1, EMB), lambda w, j, idx_s, wt_s: (idx_s[w * M + j], 0, 0))],
        out_specs=pl.BlockSpec((1, 1, EMB), lambda w, j, idx_s, wt_s: (w, 0, 0)),
        scratch_shapes=[pltpu.VMEM((1, 1, EMB), jnp.float32)],
    )
    out = pl.pallas_call(
        functools.partial(_gather_mean_body, M=M),
        grid_spec=grid_spec,
        out_shape=jax.ShapeDtypeStruct((VWP, 1, EMB), jnp.float32),
    )(idxp, wtp, table.reshape(-1, 1, EMB))
    return out.reshape(VWP, EMB)


def _lookup_body(x_ref, e1_ref, e2_ref, t0_ref, out_ref):
    xb = x_ref[...]
    oh = (xb == jax.lax.broadcasted_iota(jnp.int32, (TOK_BLOCK, VWP), 1)
          ).astype(jnp.float32)
    w1 = jnp.dot(oh, e1_ref[...], preferred_element_type=jnp.float32)
    w2 = jnp.dot(oh, e2_ref[...], preferred_element_type=jnp.float32)
    word = jnp.concatenate([w1, w2], axis=1)
    xs = jnp.minimum(xb, 3)
    ohs = (xs == jax.lax.broadcasted_iota(jnp.int32, (TOK_BLOCK, 8), 1)
           ).astype(jnp.float32)
    spec = jnp.dot(ohs, t0_ref[...], preferred_element_type=jnp.float32)
    out_ref[...] = jnp.where(xb < 4, spec, word)


def kernel(x, table0, table1, table2, idx1, mask1, cnt1, idx2, mask2, cnt2):
    B, L = x.shape
    N = B * L
    nblk = N // TOK_BLOCK

    e1 = _gather_mean(table1, idx1, mask1, cnt1, M1)
    e2 = _gather_mean(table2, idx2, mask2, cnt2, M2)
    t0p = jnp.zeros((8, 2 * EMB), jnp.float32).at[:4].set(table0)
    x3 = x.reshape(N, 1)

    out = pl.pallas_call(
        _lookup_body,
        grid=(nblk,),
        in_specs=[
            pl.BlockSpec((TOK_BLOCK, 1), lambda i: (i, 0)),
            pl.BlockSpec((VWP, EMB), lambda i: (0, 0)),
            pl.BlockSpec((VWP, EMB), lambda i: (0, 0)),
            pl.BlockSpec((8, 2 * EMB), lambda i: (0, 0)),
        ],
        out_specs=pl.BlockSpec((TOK_BLOCK, 2 * EMB), lambda i: (i, 0)),
        out_shape=jax.ShapeDtypeStruct((N, 2 * EMB), jnp.float32),
    )(x3, e1, e2, t0p)

    return out.reshape(B, L, 2 * EMB)
